# Initial kernel scaffold; baseline (speedup 1.0000x reference)
import jax
import jax.numpy as jnp
from jax.experimental import pallas as pl


def kernel(edge_index, batch, sphex, W_conv, b_conv, W_lin, b_lin):
    raise NotImplementedError("write your pallas kernel here")


# speedup vs baseline: 8.5146x; 8.5146x over previous
import functools

import jax
import jax.numpy as jnp
from jax import lax
from jax.experimental import pallas as pl
from jax.experimental.pallas import tpu as pltpu
from jax.experimental.pallas import tpu_sc as plsc

N = 10000
E = 160000
D = 256
H = D // 4
NNB = 16
NC = 2
NS = 16
CH = 125
K1C = E // (NC * NS * CH)
K3C = E // (NS * CH)
RPT = N // NS
BN = 1000
NBLK = N // BN

_mesh = functools.partial(
    plsc.VectorSubcoreMesh, core_axis_name="c", subcore_axis_name="s",
    num_cores=NC, num_subcores=NS)


def _deg_body(col3, deg_out, colv, ones_v, zero_v, deg_sp):
    c = lax.axis_index("c")
    s = lax.axis_index("s")

    ones16 = jnp.ones((16,), jnp.float32)
    zeros16 = jnp.zeros((16,), jnp.float32)

    def _fill(i, _):
        ones_v[i, :] = ones16
        zero_v[i, :] = zeros16
        return 0
    lax.fori_loop(0, CH, _fill, 0)

    for k in range(RPT // CH):
        pltpu.sync_copy(zero_v, deg_sp.at[pl.ds(s * RPT + k * CH, CH), :])
    plsc.subcore_barrier()

    pltpu.sync_copy(col3.at[c, s], colv)

    def _scat(j, _):
        pltpu.sync_copy(ones_v, deg_sp.at[colv.at[j]], add=True)
        return 0
    lax.fori_loop(0, K1C, _scat, 0)
    plsc.subcore_barrier()

    pltpu.sync_copy(deg_sp.at[pl.ds(s * RPT, RPT), :],
                    deg_out.at[c, pl.ds(s * RPT, RPT), :])


@functools.cache
def _deg_call():
    return pl.kernel(
        _deg_body,
        out_type=jax.ShapeDtypeStruct((NC, N, 16), jnp.float32),
        mesh=_mesh(),
        scratch_types=[
            pltpu.VMEM((K1C, CH), jnp.int32),
            pltpu.VMEM((CH, 16), jnp.float32),
            pltpu.VMEM((CH, 16), jnp.float32),
            pltpu.VMEM_SHARED((N, 16), jnp.float32),
        ],
        compiler_params=pltpu.CompilerParams(use_tc_tiling_on_sc=False, needs_layout_passes=False),
    )


def _dense_body(sphex_ref, wc_ref, wl_ref, bl_ref, deg0_ref, deg1_ref,
                x0_ref, x1_ref, x2_ref, x3_ref, mi_ref, dinv_ref, logz_ref,
                acc_ref):
    k = pl.program_id(0)
    f32 = jnp.float32

    sp = sphex_ref[...]
    sin = jnp.sin(sp)
    cos = jnp.cos(sp)
    u = jnp.log(jnp.maximum(jnp.abs(sin), 1e-30))
    neg = jnp.where(sin < 0, 1.0, 0.0).astype(f32)

    jj = lax.broadcasted_iota(jnp.int32, (D, D), 0)
    ii = lax.broadcasted_iota(jnp.int32, (D, D), 1)
    M = jnp.where(jj < ii, 1.0, 0.0).astype(f32)

    dn = (((1,), (0,)), ((), ()))
    prefix_log = lax.dot_general(u, M, dn, preferred_element_type=f32)
    negcnt = lax.dot_general(neg, M, dn, preferred_element_type=f32)
    parity = negcnt - 2.0 * jnp.floor(negcnt * 0.5)
    sign = 1.0 - 2.0 * parity
    colid = lax.broadcasted_iota(jnp.int32, (BN, D), 1)
    cos_part = jnp.where(colid == D - 1, 1.0, cos)
    gex = sign * jnp.exp(prefix_log) * cos_part

    dnt = (((1,), (1,)), ((), ()))
    x = lax.dot_general(gex, wc_ref[...], dnt, preferred_element_type=f32)
    deg = (jnp.sum(deg0_ref[...], axis=1, keepdims=True)
           + jnp.sum(deg1_ref[...], axis=1, keepdims=True)) * (1.0 / 16.0)
    dinv = jnp.where(deg > 0, lax.rsqrt(jnp.maximum(deg, 1e-12)), 0.0)
    xp = x * dinv
    x0_ref[...] = xp[:, :H]
    x1_ref[...] = xp[:, H:2 * H]
    x2_ref[...] = xp[:, 2 * H:3 * H]
    x3_ref[...] = xp[:, 3 * H:]
    dinv_ref[...] = dinv
    mi_ref[...] = (lax.dot_general(gex, wl_ref[...], dnt,
                                   preferred_element_type=f32) + bl_ref[...])

    colsum = jnp.sum(gex, axis=0, keepdims=True)

    @pl.when(k == 0)
    def _():
        acc_ref[...] = colsum

    @pl.when(k > 0)
    def _():
        acc_ref[...] = acc_ref[...] + colsum

    @pl.when(k == NBLK - 1)
    def _():
        m = acc_ref[...] * (1.0 / N)
        A = NNB * wc_ref[...] + 2.0 * wl_ref[...]
        v = lax.dot_general(A, m, (((1,), (1,)), ((), ())),
                            preferred_element_type=f32)
        g = jnp.sqrt(jnp.sum(v * v))
        B = wl_ref[...] + 0.5 * NNB * wc_ref[...]
        t = lax.dot_general(m, B, (((1,), (0,)), ((), ())),
                            preferred_element_type=f32)
        z_mean = -float(N) * jnp.sum(t * m)
        g_hi = jnp.maximum(g, 20.0)
        g_lo = jnp.minimum(g, 20.0)
        z_hi = float(N) * (g_hi - jnp.log(g_hi))
        z_lo = float(N) * jnp.log(
            (jnp.exp(g_lo) - jnp.exp(-g_lo)) / jnp.maximum(g_lo, 1e-30))
        z_int = jnp.where(g > 20.0, z_hi, z_lo)
        logz_ref[...] = jnp.full((1, 1), 0.0, f32) + z_mean + z_int


_dense_call = pl.pallas_call(
    _dense_body,
    grid=(NBLK,),
    in_specs=[
        pl.BlockSpec((BN, D), lambda k: (k, 0)),
        pl.BlockSpec((D, D), lambda k: (0, 0)),
        pl.BlockSpec((D, D), lambda k: (0, 0)),
        pl.BlockSpec((1, D), lambda k: (0, 0)),
        pl.BlockSpec((BN, 16), lambda k: (k, 0)),
        pl.BlockSpec((BN, 16), lambda k: (k, 0)),
    ],
    out_specs=[
        pl.BlockSpec((BN, H), lambda k: (k, 0)),
        pl.BlockSpec((BN, H), lambda k: (k, 0)),
        pl.BlockSpec((BN, H), lambda k: (k, 0)),
        pl.BlockSpec((BN, H), lambda k: (k, 0)),
        pl.BlockSpec((BN, D), lambda k: (k, 0)),
        pl.BlockSpec((BN, 1), lambda k: (k, 0)),
        pl.BlockSpec((1, 1), lambda k: (0, 0)),
    ],
    out_shape=[
        jax.ShapeDtypeStruct((N, H), jnp.float32),
        jax.ShapeDtypeStruct((N, H), jnp.float32),
        jax.ShapeDtypeStruct((N, H), jnp.float32),
        jax.ShapeDtypeStruct((N, H), jnp.float32),
        jax.ShapeDtypeStruct((N, D), jnp.float32),
        jax.ShapeDtypeStruct((N, 1), jnp.float32),
        jax.ShapeDtypeStruct((1, 1), jnp.float32),
    ],
    scratch_shapes=[pltpu.VMEM((1, D), jnp.float32)],
)


def _msg_body(row3, col3, x0, x1, x2, x3, dinv, bconv, msg0, msg1, msg2, msg3,
              rowv, colv, rbuf, dbuf, dvbuf, bcv, acc):
    c = lax.axis_index("c")
    s = lax.axis_index("s")
    base = s * RPT

    zeros16 = jnp.zeros((16,), jnp.float32)

    pltpu.sync_copy(row3.at[s], rowv)
    pltpu.sync_copy(col3.at[s], colv)

    def _pass(xq, msgq, q):
        def _zrow(i, _):
            for v in range(H // 16):
                dbuf[i, pl.ds(16 * v, 16)] = zeros16
            return 0
        lax.fori_loop(0, CH, _zrow, 0)
        for k in range(RPT // CH):
            pltpu.sync_copy(dbuf, acc.at[pl.ds(base + k * CH, CH), :])
        pltpu.sync_copy(bconv.at[pl.ds(H * q, H)], bcv)
        plsc.subcore_barrier()

        def _step(j, _):
            pltpu.sync_copy(xq.at[rowv.at[j]], rbuf)
            pltpu.sync_copy(rbuf, acc.at[colv.at[j]], add=True)
            return 0
        lax.fori_loop(0, K3C, _step, 0)
        plsc.subcore_barrier()

        for k in range(RPT // CH):
            r0 = base + k * CH
            pltpu.sync_copy(acc.at[pl.ds(r0, CH), :], dbuf)
            pltpu.sync_copy(dinv.at[pl.ds(r0, CH), :], dvbuf)

            def _scale(i, _):
                dv = plsc.load_gather(
                    dvbuf, [jnp.full((16,), i, jnp.int32),
                            jnp.zeros((16,), jnp.int32)])
                for v in range(H // 16):
                    sl = pl.ds(16 * v, 16)
                    dbuf[i, sl] = dbuf[i, sl] * dv + bcv[sl]
                return 0
            lax.fori_loop(0, CH, _scale, 0)
            pltpu.sync_copy(dbuf, msgq.at[pl.ds(r0, CH), :])

    @pl.when(c == 0)
    def _():
        _pass(x0, msg0, 0)
        _pass(x1, msg1, 1)

    @pl.when(c == 1)
    def _():
        _pass(x2, msg2, 2)
        _pass(x3, msg3, 3)


@functools.cache
def _msg_call():
    return pl.kernel(
        _msg_body,
        out_type=tuple(jax.ShapeDtypeStruct((N, H), jnp.float32)
                       for _ in range(4)),
        mesh=_mesh(),
        scratch_types=[
            pltpu.VMEM((K3C, CH), jnp.int32),
            pltpu.VMEM((K3C, CH), jnp.int32),
            pltpu.VMEM((CH, H), jnp.float32),
            pltpu.VMEM((CH, H), jnp.float32),
            pltpu.VMEM((CH, 1), jnp.float32),
            pltpu.VMEM((H,), jnp.float32),
            pltpu.VMEM_SHARED((N, H), jnp.float32),
        ],
        compiler_params=pltpu.CompilerParams(use_tc_tiling_on_sc=False, needs_layout_passes=False),
    )


def kernel(edge_index, batch, sphex, W_conv, b_conv, W_lin, b_lin):
    del batch
    row = edge_index[0]
    col = edge_index[1]
    col_k1 = col.reshape(NC, NS, K1C, CH)
    row_k3 = row.reshape(NS, K3C, CH)
    col_k3 = col.reshape(NS, K3C, CH)

    deg_parts = _deg_call()(col_k1)

    sphex_pad = jnp.pad(sphex, ((0, 0), (0, 1)))
    x0, x1, x2, x3, msg_intra, dinv, logz = _dense_call(
        sphex_pad, W_conv, W_lin, b_lin.reshape(1, D),
        deg_parts[0], deg_parts[1])

    msg0, msg1, msg2, msg3 = _msg_call()(
        row_k3, col_k3, x0, x1, x2, x3, dinv, b_conv)
    msg = jnp.concatenate([msg0, msg1, msg2, msg3], axis=1)
    return (msg, msg_intra, logz)


# speedup vs baseline: 11.1242x; 1.3065x over previous
import functools

import jax
import jax.numpy as jnp
from jax import lax
from jax.experimental import pallas as pl
from jax.experimental.pallas import tpu as pltpu
from jax.experimental.pallas import tpu_sc as plsc

N = 10000
E = 160000
D = 256
H = D // 4
NNB = 16
NC = 2
NS = 16
CH = 125
K1C = E // (NC * NS * CH)
K3C = E // (NS * CH)
RPT = N // NS
BN = 1000
NBLK = N // BN

_mesh = functools.partial(
    plsc.VectorSubcoreMesh, core_axis_name="c", subcore_axis_name="s",
    num_cores=NC, num_subcores=NS)


def _deg_body(col3, deg_out, colv, ones_v, zero_v, deg_sp):
    c = lax.axis_index("c")
    s = lax.axis_index("s")

    ones16 = jnp.ones((16,), jnp.float32)
    zeros16 = jnp.zeros((16,), jnp.float32)

    def _fill(i, _):
        ones_v[i, :] = ones16
        zero_v[i, :] = zeros16
        return 0
    lax.fori_loop(0, CH, _fill, 0)

    for k in range(RPT // CH):
        pltpu.sync_copy(zero_v, deg_sp.at[pl.ds(s * RPT + k * CH, CH), :])
    plsc.subcore_barrier()

    pltpu.sync_copy(col3.at[c, s], colv)

    def _scat(j, _):
        pltpu.sync_copy(ones_v, deg_sp.at[colv.at[j]], add=True)
        return 0
    lax.fori_loop(0, K1C, _scat, 0)
    plsc.subcore_barrier()

    pltpu.sync_copy(deg_sp.at[pl.ds(s * RPT, RPT), :],
                    deg_out.at[c, pl.ds(s * RPT, RPT), :])


@functools.cache
def _deg_call():
    return pl.kernel(
        _deg_body,
        out_type=jax.ShapeDtypeStruct((NC, N, 16), jnp.float32),
        mesh=_mesh(),
        scratch_types=[
            pltpu.VMEM((K1C, CH), jnp.int32),
            pltpu.VMEM((CH, 16), jnp.float32),
            pltpu.VMEM((CH, 16), jnp.float32),
            pltpu.VMEM_SHARED((N, 16), jnp.float32),
        ],
        compiler_params=pltpu.CompilerParams(use_tc_tiling_on_sc=False, needs_layout_passes=False),
    )


def _dense_body(sphex_ref, wc_ref, wl_ref, bl_ref, deg0_ref, deg1_ref,
                x0_ref, x1_ref, x2_ref, x3_ref, mi_ref, dinv_ref, logz_ref,
                acc_ref):
    k = pl.program_id(0)
    f32 = jnp.float32

    sp = sphex_ref[...]
    sin = jnp.sin(sp)
    cos = jnp.cos(sp)
    u = jnp.log(jnp.maximum(jnp.abs(sin), 1e-30))
    neg = jnp.where(sin < 0, 1.0, 0.0).astype(f32)

    jj = lax.broadcasted_iota(jnp.int32, (D, D), 0)
    ii = lax.broadcasted_iota(jnp.int32, (D, D), 1)
    M = jnp.where(jj < ii, 1.0, 0.0).astype(f32)

    dn = (((1,), (0,)), ((), ()))
    prefix_log = lax.dot_general(u, M, dn, preferred_element_type=f32)
    negcnt = lax.dot_general(neg, M, dn, preferred_element_type=f32)
    parity = negcnt - 2.0 * jnp.floor(negcnt * 0.5)
    sign = 1.0 - 2.0 * parity
    colid = lax.broadcasted_iota(jnp.int32, (BN, D), 1)
    cos_part = jnp.where(colid == D - 1, 1.0, cos)
    gex = sign * jnp.exp(prefix_log) * cos_part

    dnt = (((1,), (1,)), ((), ()))
    x = lax.dot_general(gex, wc_ref[...], dnt, preferred_element_type=f32)
    deg = (jnp.sum(deg0_ref[...], axis=1, keepdims=True)
           + jnp.sum(deg1_ref[...], axis=1, keepdims=True)) * (1.0 / 16.0)
    dinv = jnp.where(deg > 0, lax.rsqrt(jnp.maximum(deg, 1e-12)), 0.0)
    xp = x * dinv
    x0_ref[...] = xp[:, :H]
    x1_ref[...] = xp[:, H:2 * H]
    x2_ref[...] = xp[:, 2 * H:3 * H]
    x3_ref[...] = xp[:, 3 * H:]
    dinv_ref[...] = dinv
    mi_ref[...] = (lax.dot_general(gex, wl_ref[...], dnt,
                                   preferred_element_type=f32) + bl_ref[...])

    colsum = jnp.sum(gex, axis=0, keepdims=True)

    @pl.when(k == 0)
    def _():
        acc_ref[...] = colsum

    @pl.when(k > 0)
    def _():
        acc_ref[...] = acc_ref[...] + colsum

    @pl.when(k == NBLK - 1)
    def _():
        m = acc_ref[...] * (1.0 / N)
        A = NNB * wc_ref[...] + 2.0 * wl_ref[...]
        v = lax.dot_general(A, m, (((1,), (1,)), ((), ())),
                            preferred_element_type=f32)
        g = jnp.sqrt(jnp.sum(v * v))
        B = wl_ref[...] + 0.5 * NNB * wc_ref[...]
        t = lax.dot_general(m, B, (((1,), (0,)), ((), ())),
                            preferred_element_type=f32)
        z_mean = -float(N) * jnp.sum(t * m)
        g_hi = jnp.maximum(g, 20.0)
        g_lo = jnp.minimum(g, 20.0)
        z_hi = float(N) * (g_hi - jnp.log(g_hi))
        z_lo = float(N) * jnp.log(
            (jnp.exp(g_lo) - jnp.exp(-g_lo)) / jnp.maximum(g_lo, 1e-30))
        z_int = jnp.where(g > 20.0, z_hi, z_lo)
        logz_ref[...] = jnp.full((1, 1), 0.0, f32) + z_mean + z_int


_dense_call = pl.pallas_call(
    _dense_body,
    grid=(NBLK,),
    in_specs=[
        pl.BlockSpec((BN, D), lambda k: (k, 0)),
        pl.BlockSpec((D, D), lambda k: (0, 0)),
        pl.BlockSpec((D, D), lambda k: (0, 0)),
        pl.BlockSpec((1, D), lambda k: (0, 0)),
        pl.BlockSpec((BN, 16), lambda k: (k, 0)),
        pl.BlockSpec((BN, 16), lambda k: (k, 0)),
    ],
    out_specs=[
        pl.BlockSpec((BN, H), lambda k: (k, 0)),
        pl.BlockSpec((BN, H), lambda k: (k, 0)),
        pl.BlockSpec((BN, H), lambda k: (k, 0)),
        pl.BlockSpec((BN, H), lambda k: (k, 0)),
        pl.BlockSpec((BN, D), lambda k: (k, 0)),
        pl.BlockSpec((BN, 1), lambda k: (k, 0)),
        pl.BlockSpec((1, 1), lambda k: (0, 0)),
    ],
    out_shape=[
        jax.ShapeDtypeStruct((N, H), jnp.float32),
        jax.ShapeDtypeStruct((N, H), jnp.float32),
        jax.ShapeDtypeStruct((N, H), jnp.float32),
        jax.ShapeDtypeStruct((N, H), jnp.float32),
        jax.ShapeDtypeStruct((N, D), jnp.float32),
        jax.ShapeDtypeStruct((N, 1), jnp.float32),
        jax.ShapeDtypeStruct((1, 1), jnp.float32),
    ],
    scratch_shapes=[pltpu.VMEM((1, D), jnp.float32)],
)


def _msg_body(row3, col3, x0, x1, x2, x3, dinv, bconv, msg0, msg1, msg2, msg3,
              rowv, colv, rbuf, rbuf2, dbuf, dvbuf, bcv, sem0, sem1, acc):
    c = lax.axis_index("c")
    s = lax.axis_index("s")
    base = s * RPT

    zeros16 = jnp.zeros((16,), jnp.float32)

    pltpu.sync_copy(row3.at[s], rowv)
    pltpu.sync_copy(col3.at[s], colv)

    def _pass(xq, msgq, q):
        def _zrow(i, _):
            for v in range(H // 16):
                dbuf[i, pl.ds(16 * v, 16)] = zeros16
            return 0
        lax.fori_loop(0, CH, _zrow, 0)
        for k in range(RPT // CH):
            pltpu.sync_copy(dbuf, acc.at[pl.ds(base + k * CH, CH), :])
        pltpu.sync_copy(bconv.at[pl.ds(H * q, H)], bcv)
        plsc.subcore_barrier()

        cp0 = pltpu.async_copy(xq.at[rowv.at[0]], rbuf, sem0)
        cp1 = pltpu.async_copy(xq.at[rowv.at[1]], rbuf2, sem1)

        def _step2(g, _):
            j = 2 * g
            pltpu.make_async_copy(xq.at[rowv.at[j]], rbuf, sem0).wait()
            pltpu.sync_copy(rbuf, acc.at[colv.at[j]], add=True)

            @pl.when(j + 2 < K3C)
            def _():
                pltpu.async_copy(xq.at[rowv.at[j + 2]], rbuf, sem0)

            pltpu.make_async_copy(xq.at[rowv.at[j + 1]], rbuf2, sem1).wait()
            pltpu.sync_copy(rbuf2, acc.at[colv.at[j + 1]], add=True)

            @pl.when(j + 3 < K3C)
            def _():
                pltpu.async_copy(xq.at[rowv.at[j + 3]], rbuf2, sem1)

            return 0
        lax.fori_loop(0, K3C // 2, _step2, 0)
        plsc.subcore_barrier()

        for k in range(RPT // CH):
            r0 = base + k * CH
            pltpu.sync_copy(acc.at[pl.ds(r0, CH), :], dbuf)
            pltpu.sync_copy(dinv.at[pl.ds(r0, CH), :], dvbuf)

            def _scale(i, _):
                dv = plsc.load_gather(
                    dvbuf, [jnp.full((16,), i, jnp.int32),
                            jnp.zeros((16,), jnp.int32)])
                for v in range(H // 16):
                    sl = pl.ds(16 * v, 16)
                    dbuf[i, sl] = dbuf[i, sl] * dv + bcv[sl]
                return 0
            lax.fori_loop(0, CH, _scale, 0)
            pltpu.sync_copy(dbuf, msgq.at[pl.ds(r0, CH), :])

    @pl.when(c == 0)
    def _():
        _pass(x0, msg0, 0)
        _pass(x1, msg1, 1)

    @pl.when(c == 1)
    def _():
        _pass(x2, msg2, 2)
        _pass(x3, msg3, 3)


@functools.cache
def _msg_call():
    return pl.kernel(
        _msg_body,
        out_type=tuple(jax.ShapeDtypeStruct((N, H), jnp.float32)
                       for _ in range(4)),
        mesh=_mesh(),
        scratch_types=[
            pltpu.VMEM((K3C, CH), jnp.int32),
            pltpu.VMEM((K3C, CH), jnp.int32),
            pltpu.VMEM((CH, H), jnp.float32),
            pltpu.VMEM((CH, H), jnp.float32),
            pltpu.VMEM((CH, H), jnp.float32),
            pltpu.VMEM((CH, 1), jnp.float32),
            pltpu.VMEM((H,), jnp.float32),
            pltpu.SemaphoreType.DMA,
            pltpu.SemaphoreType.DMA,
            pltpu.VMEM_SHARED((N, H), jnp.float32),
        ],
        compiler_params=pltpu.CompilerParams(use_tc_tiling_on_sc=False, needs_layout_passes=False),
    )


def kernel(edge_index, batch, sphex, W_conv, b_conv, W_lin, b_lin):
    del batch
    row = edge_index[0]
    col = edge_index[1]
    col_k1 = col.reshape(NC, NS, K1C, CH)
    row_k3 = row.reshape(NS, K3C, CH)
    col_k3 = col.reshape(NS, K3C, CH)

    deg_parts = _deg_call()(col_k1)

    sphex_pad = jnp.pad(sphex, ((0, 0), (0, 1)))
    x0, x1, x2, x3, msg_intra, dinv, logz = _dense_call(
        sphex_pad, W_conv, W_lin, b_lin.reshape(1, D),
        deg_parts[0], deg_parts[1])

    msg0, msg1, msg2, msg3 = _msg_call()(
        row_k3, col_k3, x0, x1, x2, x3, dinv, b_conv)
    msg = jnp.concatenate([msg0, msg1, msg2, msg3], axis=1)
    return (msg, msg_intra, logz)


# speedup vs baseline: 11.9040x; 1.0701x over previous
import functools

import jax
import jax.numpy as jnp
from jax import lax
from jax.experimental import pallas as pl
from jax.experimental.pallas import tpu as pltpu
from jax.experimental.pallas import tpu_sc as plsc

N = 10000
E = 160000
D = 256
H = D // 4
NNB = 16
NC = 2
NS = 16
CH = 125
K1C = E // (NC * NS * CH)
K3C = E // (NS * CH)
RPT = N // NS
BN = 1000
NBLK = N // BN

_mesh = functools.partial(
    plsc.VectorSubcoreMesh, core_axis_name="c", subcore_axis_name="s",
    num_cores=NC, num_subcores=NS)


def _deg_body(col3, deg_out, colv, ones_v, zero_v, deg_sp):
    c = lax.axis_index("c")
    s = lax.axis_index("s")

    ones16 = jnp.ones((16,), jnp.float32)
    zeros16 = jnp.zeros((16,), jnp.float32)

    def _fill(i, _):
        ones_v[i, :] = ones16
        zero_v[i, :] = zeros16
        return 0
    lax.fori_loop(0, CH, _fill, 0)

    for k in range(RPT // CH):
        pltpu.sync_copy(zero_v, deg_sp.at[pl.ds(s * RPT + k * CH, CH), :])
    plsc.subcore_barrier()

    pltpu.sync_copy(col3.at[c, s], colv)

    def _scat(j, _):
        pltpu.sync_copy(ones_v, deg_sp.at[colv.at[j]], add=True)
        return 0
    lax.fori_loop(0, K1C, _scat, 0)
    plsc.subcore_barrier()

    pltpu.sync_copy(deg_sp.at[pl.ds(s * RPT, RPT), :],
                    deg_out.at[c, pl.ds(s * RPT, RPT), :])


@functools.cache
def _deg_call():
    return pl.kernel(
        _deg_body,
        out_type=jax.ShapeDtypeStruct((NC, N, 16), jnp.float32),
        mesh=_mesh(),
        scratch_types=[
            pltpu.VMEM((K1C, CH), jnp.int32),
            pltpu.VMEM((CH, 16), jnp.float32),
            pltpu.VMEM((CH, 16), jnp.float32),
            pltpu.VMEM_SHARED((N, 16), jnp.float32),
        ],
        compiler_params=pltpu.CompilerParams(use_tc_tiling_on_sc=False, needs_layout_passes=False),
    )


def _dense_body(sphex_ref, wc_ref, wl_ref, bl_ref, deg0_ref, deg1_ref,
                x0_ref, x1_ref, x2_ref, x3_ref, mi_ref, dinv_ref, logz_ref,
                acc_ref):
    k = pl.program_id(0)
    f32 = jnp.float32

    sp = sphex_ref[...]
    sin = jnp.sin(sp)
    cos = jnp.cos(sp)
    u = jnp.log(jnp.maximum(jnp.abs(sin), 1e-30))
    neg = jnp.where(sin < 0, 1.0, 0.0).astype(f32)

    jj = lax.broadcasted_iota(jnp.int32, (D, D), 0)
    ii = lax.broadcasted_iota(jnp.int32, (D, D), 1)
    M = jnp.where(jj < ii, 1.0, 0.0).astype(f32)

    dn = (((1,), (0,)), ((), ()))
    prefix_log = lax.dot_general(u, M, dn, preferred_element_type=f32)
    negcnt = lax.dot_general(neg, M, dn, preferred_element_type=f32)
    parity = negcnt - 2.0 * jnp.floor(negcnt * 0.5)
    sign = 1.0 - 2.0 * parity
    colid = lax.broadcasted_iota(jnp.int32, (BN, D), 1)
    cos_part = jnp.where(colid == D - 1, 1.0, cos)
    gex = sign * jnp.exp(prefix_log) * cos_part

    dnt = (((1,), (1,)), ((), ()))
    x = lax.dot_general(gex, wc_ref[...], dnt, preferred_element_type=f32)
    deg = (jnp.sum(deg0_ref[...], axis=1, keepdims=True)
           + jnp.sum(deg1_ref[...], axis=1, keepdims=True)) * (1.0 / 16.0)
    dinv = jnp.where(deg > 0, lax.rsqrt(jnp.maximum(deg, 1e-12)), 0.0)
    xp = x * dinv
    x0_ref[...] = xp[:, :H]
    x1_ref[...] = xp[:, H:2 * H]
    x2_ref[...] = xp[:, 2 * H:3 * H]
    x3_ref[...] = xp[:, 3 * H:]
    dinv_ref[...] = dinv
    mi_ref[...] = (lax.dot_general(gex, wl_ref[...], dnt,
                                   preferred_element_type=f32) + bl_ref[...])

    colsum = jnp.sum(gex, axis=0, keepdims=True)

    @pl.when(k == 0)
    def _():
        acc_ref[...] = colsum

    @pl.when(k > 0)
    def _():
        acc_ref[...] = acc_ref[...] + colsum

    @pl.when(k == NBLK - 1)
    def _():
        m = acc_ref[...] * (1.0 / N)
        A = NNB * wc_ref[...] + 2.0 * wl_ref[...]
        v = lax.dot_general(A, m, (((1,), (1,)), ((), ())),
                            preferred_element_type=f32)
        g = jnp.sqrt(jnp.sum(v * v))
        B = wl_ref[...] + 0.5 * NNB * wc_ref[...]
        t = lax.dot_general(m, B, (((1,), (0,)), ((), ())),
                            preferred_element_type=f32)
        z_mean = -float(N) * jnp.sum(t * m)
        g_hi = jnp.maximum(g, 20.0)
        g_lo = jnp.minimum(g, 20.0)
        z_hi = float(N) * (g_hi - jnp.log(g_hi))
        z_lo = float(N) * jnp.log(
            (jnp.exp(g_lo) - jnp.exp(-g_lo)) / jnp.maximum(g_lo, 1e-30))
        z_int = jnp.where(g > 20.0, z_hi, z_lo)
        logz_ref[...] = jnp.full((1, 1), 0.0, f32) + z_mean + z_int


_dense_call = pl.pallas_call(
    _dense_body,
    grid=(NBLK,),
    in_specs=[
        pl.BlockSpec((BN, D), lambda k: (k, 0)),
        pl.BlockSpec((D, D), lambda k: (0, 0)),
        pl.BlockSpec((D, D), lambda k: (0, 0)),
        pl.BlockSpec((1, D), lambda k: (0, 0)),
        pl.BlockSpec((BN, 16), lambda k: (k, 0)),
        pl.BlockSpec((BN, 16), lambda k: (k, 0)),
    ],
    out_specs=[
        pl.BlockSpec((BN, H), lambda k: (k, 0)),
        pl.BlockSpec((BN, H), lambda k: (k, 0)),
        pl.BlockSpec((BN, H), lambda k: (k, 0)),
        pl.BlockSpec((BN, H), lambda k: (k, 0)),
        pl.BlockSpec((BN, D), lambda k: (k, 0)),
        pl.BlockSpec((BN, 1), lambda k: (k, 0)),
        pl.BlockSpec((1, 1), lambda k: (0, 0)),
    ],
    out_shape=[
        jax.ShapeDtypeStruct((N, H), jnp.float32),
        jax.ShapeDtypeStruct((N, H), jnp.float32),
        jax.ShapeDtypeStruct((N, H), jnp.float32),
        jax.ShapeDtypeStruct((N, H), jnp.float32),
        jax.ShapeDtypeStruct((N, D), jnp.float32),
        jax.ShapeDtypeStruct((N, 1), jnp.float32),
        jax.ShapeDtypeStruct((1, 1), jnp.float32),
    ],
    scratch_shapes=[pltpu.VMEM((1, D), jnp.float32)],
)


def _msg_body(row3, col3, x0, x1, x2, x3, dinv, bconv, msg0, msg1, msg2, msg3,
              rowv, colv, rb0, rb1, rb2, rb3, dbuf, dvbuf, bcv,
              sg0, sg1, sg2, sg3, ss0, ss1, ss2, ss3, acc):
    c = lax.axis_index("c")
    s = lax.axis_index("s")
    base = s * RPT

    zeros16 = jnp.zeros((16,), jnp.float32)

    pltpu.sync_copy(row3.at[s], rowv)
    pltpu.sync_copy(col3.at[s], colv)

    def _pass(xq, msgq, q):
        def _zrow(i, _):
            for v in range(H // 16):
                dbuf[i, pl.ds(16 * v, 16)] = zeros16
            return 0
        lax.fori_loop(0, CH, _zrow, 0)
        for k in range(RPT // CH):
            pltpu.sync_copy(dbuf, acc.at[pl.ds(base + k * CH, CH), :])
        pltpu.sync_copy(bconv.at[pl.ds(H * q, H)], bcv)
        plsc.subcore_barrier()

        rbufs = (rb0, rb1, rb2, rb3)
        sgs = (sg0, sg1, sg2, sg3)
        sss = (ss0, ss1, ss2, ss3)
        pltpu.async_copy(xq.at[rowv.at[0]], rbufs[0], sgs[0])
        pltpu.async_copy(xq.at[rowv.at[1]], rbufs[1], sgs[1])

        def _slot(g, _):
            for b in range(4):
                j = 4 * g + b
                b2 = (b + 2) % 4

                @pl.when(j >= 2)
                def _():
                    pltpu.make_async_copy(
                        rbufs[b2], acc.at[colv.at[j - 2]], sss[b2]).wait()

                @pl.when(j + 2 < K3C)
                def _():
                    pltpu.async_copy(
                        xq.at[rowv.at[j + 2]], rbufs[b2], sgs[b2])

                pltpu.make_async_copy(
                    xq.at[rowv.at[j]], rbufs[b], sgs[b]).wait()
                pltpu.async_copy(
                    rbufs[b], acc.at[colv.at[j]], sss[b], add=True)
            return 0
        lax.fori_loop(0, K3C // 4, _slot, 0)
        pltpu.make_async_copy(
            rbufs[(K3C - 2) % 4], acc.at[colv.at[K3C - 2]],
            sss[(K3C - 2) % 4]).wait()
        pltpu.make_async_copy(
            rbufs[(K3C - 1) % 4], acc.at[colv.at[K3C - 1]],
            sss[(K3C - 1) % 4]).wait()
        plsc.subcore_barrier()

        for k in range(RPT // CH):
            r0 = base + k * CH
            pltpu.sync_copy(acc.at[pl.ds(r0, CH), :], dbuf)
            pltpu.sync_copy(dinv.at[pl.ds(r0, CH), :], dvbuf)

            def _scale(i, _):
                dv = plsc.load_gather(
                    dvbuf, [jnp.full((16,), i, jnp.int32),
                            jnp.zeros((16,), jnp.int32)])
                for v in range(H // 16):
                    sl = pl.ds(16 * v, 16)
                    dbuf[i, sl] = dbuf[i, sl] * dv + bcv[sl]
                return 0
            lax.fori_loop(0, CH, _scale, 0)
            pltpu.sync_copy(dbuf, msgq.at[pl.ds(r0, CH), :])

    @pl.when(c == 0)
    def _():
        _pass(x0, msg0, 0)
        _pass(x1, msg1, 1)

    @pl.when(c == 1)
    def _():
        _pass(x2, msg2, 2)
        _pass(x3, msg3, 3)


@functools.cache
def _msg_call():
    return pl.kernel(
        _msg_body,
        out_type=tuple(jax.ShapeDtypeStruct((N, H), jnp.float32)
                       for _ in range(4)),
        mesh=_mesh(),
        scratch_types=[
            pltpu.VMEM((K3C, CH), jnp.int32),
            pltpu.VMEM((K3C, CH), jnp.int32),
            pltpu.VMEM((CH, H), jnp.float32),
            pltpu.VMEM((CH, H), jnp.float32),
            pltpu.VMEM((CH, H), jnp.float32),
            pltpu.VMEM((CH, H), jnp.float32),
            pltpu.VMEM((CH, H), jnp.float32),
            pltpu.VMEM((CH, 1), jnp.float32),
            pltpu.VMEM((H,), jnp.float32),
            pltpu.SemaphoreType.DMA,
            pltpu.SemaphoreType.DMA,
            pltpu.SemaphoreType.DMA,
            pltpu.SemaphoreType.DMA,
            pltpu.SemaphoreType.DMA,
            pltpu.SemaphoreType.DMA,
            pltpu.SemaphoreType.DMA,
            pltpu.SemaphoreType.DMA,
            pltpu.VMEM_SHARED((N, H), jnp.float32),
        ],
        compiler_params=pltpu.CompilerParams(use_tc_tiling_on_sc=False, needs_layout_passes=False),
    )


def kernel(edge_index, batch, sphex, W_conv, b_conv, W_lin, b_lin):
    del batch
    row = edge_index[0]
    col = edge_index[1]
    col_k1 = col.reshape(NC, NS, K1C, CH)
    row_k3 = row.reshape(NS, K3C, CH)
    col_k3 = col.reshape(NS, K3C, CH)

    deg_parts = _deg_call()(col_k1)

    sphex_pad = jnp.pad(sphex, ((0, 0), (0, 1)))
    x0, x1, x2, x3, msg_intra, dinv, logz = _dense_call(
        sphex_pad, W_conv, W_lin, b_lin.reshape(1, D),
        deg_parts[0], deg_parts[1])

    msg0, msg1, msg2, msg3 = _msg_call()(
        row_k3, col_k3, x0, x1, x2, x3, dinv, b_conv)
    msg = jnp.concatenate([msg0, msg1, msg2, msg3], axis=1)
    return (msg, msg_intra, logz)


# speedup vs baseline: 12.8479x; 1.0793x over previous
import functools

import jax
import jax.numpy as jnp
from jax import lax
from jax.experimental import pallas as pl
from jax.experimental.pallas import tpu as pltpu
from jax.experimental.pallas import tpu_sc as plsc

N = 10000
E = 160000
D = 256
H = D // 4
NNB = 16
NC = 2
NS = 16
CH = 125
K1C = E // (NC * NS * CH)
K3C = E // (NS * CH)
RPT = N // NS
BN = 1000
NBLK = N // BN

_mesh = functools.partial(
    plsc.VectorSubcoreMesh, core_axis_name="c", subcore_axis_name="s",
    num_cores=NC, num_subcores=NS)


def _deg_body(col3, deg_out, colv, ones_v, zero_v, dchunk, dcomp, deg_sp):
    c = lax.axis_index("c")
    s = lax.axis_index("s")

    ones16 = jnp.ones((16,), jnp.float32)
    zeros16 = jnp.zeros((16,), jnp.float32)

    def _fill(i, _):
        ones_v[i, :] = ones16
        zero_v[i, :] = zeros16
        return 0
    lax.fori_loop(0, CH, _fill, 0)

    for k in range(RPT // CH):
        pltpu.sync_copy(zero_v, deg_sp.at[pl.ds(s * RPT + k * CH, CH), :])
    plsc.subcore_barrier()

    pltpu.sync_copy(col3.at[c, s], colv)

    def _scat(j, _):
        pltpu.sync_copy(ones_v, deg_sp.at[colv.at[j]], add=True)
        return 0
    lax.fori_loop(0, K1C, _scat, 0)
    plsc.subcore_barrier()

    pltpu.sync_copy(deg_sp.at[pl.ds(s * RPT, RPT), :], dchunk)
    i16 = lax.broadcasted_iota(jnp.int32, (16,), 0)
    z16 = jnp.zeros((16,), jnp.int32)
    for st in [16 * g for g in range(RPT // 16)] + [RPT - 16]:
        dcomp[pl.ds(st, 16)] = plsc.load_gather(dchunk, [i16 + st, z16])
    pltpu.sync_copy(dcomp, deg_out.at[c, s])


@functools.cache
def _deg_call():
    return pl.kernel(
        _deg_body,
        out_type=jax.ShapeDtypeStruct((NC, NS, RPT), jnp.float32),
        mesh=_mesh(),
        scratch_types=[
            pltpu.VMEM((K1C, CH), jnp.int32),
            pltpu.VMEM((CH, 16), jnp.float32),
            pltpu.VMEM((CH, 16), jnp.float32),
            pltpu.VMEM((RPT, 16), jnp.float32),
            pltpu.VMEM((RPT,), jnp.float32),
            pltpu.VMEM_SHARED((N, 16), jnp.float32),
        ],
        compiler_params=pltpu.CompilerParams(use_tc_tiling_on_sc=False, needs_layout_passes=False),
    )


def _dense_body(sphex_ref, wc_ref, wl_ref, bl_ref, deg_ref,
                x0_ref, x1_ref, x2_ref, x3_ref, mi_ref, logz_ref,
                acc_ref):
    k = pl.program_id(0)
    f32 = jnp.float32

    sp = sphex_ref[...]
    sin = jnp.sin(sp)
    cos = jnp.cos(sp)
    u = jnp.log(jnp.maximum(jnp.abs(sin), 1e-30))
    neg = jnp.where(sin < 0, 1.0, 0.0).astype(f32)

    jj = lax.broadcasted_iota(jnp.int32, (D, D), 0)
    ii = lax.broadcasted_iota(jnp.int32, (D, D), 1)
    M = jnp.where(jj < ii, 1.0, 0.0).astype(f32)

    dn = (((1,), (0,)), ((), ()))
    prefix_log = lax.dot_general(u, M, dn, preferred_element_type=f32)
    negcnt = lax.dot_general(neg, M, dn, preferred_element_type=f32)
    parity = negcnt - 2.0 * jnp.floor(negcnt * 0.5)
    sign = 1.0 - 2.0 * parity
    colid = lax.broadcasted_iota(jnp.int32, (BN, D), 1)
    cos_part = jnp.where(colid == D - 1, 1.0, cos)
    gex = sign * jnp.exp(prefix_log) * cos_part

    dnt = (((1,), (1,)), ((), ()))
    x = lax.dot_general(gex, wc_ref[...], dnt, preferred_element_type=f32)
    dblk = deg_ref[0]
    deg = dblk[0:1, :] + dblk[1:2, :]
    dinv_row = jnp.where(deg > 0, lax.rsqrt(jnp.maximum(deg, 1e-12)), 0.0)
    dinv = lax.dot_general(dinv_row, jnp.ones((1, 1), f32),
                           (((0,), (0,)), ((), ())),
                           preferred_element_type=f32)
    xp = x * dinv
    x0_ref[...] = xp[:, :H]
    x1_ref[...] = xp[:, H:2 * H]
    x2_ref[...] = xp[:, 2 * H:3 * H]
    x3_ref[...] = xp[:, 3 * H:]
    mi_ref[...] = (lax.dot_general(gex, wl_ref[...], dnt,
                                   preferred_element_type=f32) + bl_ref[...])

    colsum = jnp.sum(gex, axis=0, keepdims=True)

    @pl.when(k == 0)
    def _():
        acc_ref[...] = colsum

    @pl.when(k > 0)
    def _():
        acc_ref[...] = acc_ref[...] + colsum

    @pl.when(k == NBLK - 1)
    def _():
        m = acc_ref[...] * (1.0 / N)
        A = NNB * wc_ref[...] + 2.0 * wl_ref[...]
        v = lax.dot_general(A, m, (((1,), (1,)), ((), ())),
                            preferred_element_type=f32)
        g = jnp.sqrt(jnp.sum(v * v))
        B = wl_ref[...] + 0.5 * NNB * wc_ref[...]
        t = lax.dot_general(m, B, (((1,), (0,)), ((), ())),
                            preferred_element_type=f32)
        z_mean = -float(N) * jnp.sum(t * m)
        g_hi = jnp.maximum(g, 20.0)
        g_lo = jnp.minimum(g, 20.0)
        z_hi = float(N) * (g_hi - jnp.log(g_hi))
        z_lo = float(N) * jnp.log(
            (jnp.exp(g_lo) - jnp.exp(-g_lo)) / jnp.maximum(g_lo, 1e-30))
        z_int = jnp.where(g > 20.0, z_hi, z_lo)
        logz_ref[...] = jnp.full((1, 1), 0.0, f32) + z_mean + z_int


_dense_call = pl.pallas_call(
    _dense_body,
    grid=(NBLK,),
    in_specs=[
        pl.BlockSpec((BN, D), lambda k: (k, 0)),
        pl.BlockSpec((D, D), lambda k: (0, 0)),
        pl.BlockSpec((D, D), lambda k: (0, 0)),
        pl.BlockSpec((1, D), lambda k: (0, 0)),
        pl.BlockSpec((1, 2, BN), lambda k: (k, 0, 0)),
    ],
    out_specs=[
        pl.BlockSpec((BN, H), lambda k: (k, 0)),
        pl.BlockSpec((BN, H), lambda k: (k, 0)),
        pl.BlockSpec((BN, H), lambda k: (k, 0)),
        pl.BlockSpec((BN, H), lambda k: (k, 0)),
        pl.BlockSpec((BN, D), lambda k: (k, 0)),
        pl.BlockSpec((1, 1), lambda k: (0, 0)),
    ],
    out_shape=[
        jax.ShapeDtypeStruct((N, H), jnp.float32),
        jax.ShapeDtypeStruct((N, H), jnp.float32),
        jax.ShapeDtypeStruct((N, H), jnp.float32),
        jax.ShapeDtypeStruct((N, H), jnp.float32),
        jax.ShapeDtypeStruct((N, D), jnp.float32),
        jax.ShapeDtypeStruct((1, 1), jnp.float32),
    ],
    scratch_shapes=[pltpu.VMEM((1, D), jnp.float32)],
)


def _msg_body(row3, col3, x0, x1, x2, x3, degc, bconv, msg0, msg1, msg2, msg3,
              rowv, colv, rb0, rb1, rb2, rb3, dbuf, dva, dvb, dinv_v, bcv,
              sg0, sg1, sg2, sg3, ss0, ss1, ss2, ss3, acc):
    c = lax.axis_index("c")
    s = lax.axis_index("s")
    base = s * RPT

    zeros16 = jnp.zeros((16,), jnp.float32)

    pltpu.sync_copy(row3.at[s], rowv)
    pltpu.sync_copy(col3.at[s], colv)

    pltpu.sync_copy(degc.at[0, s], dva)
    pltpu.sync_copy(degc.at[1, s], dvb)
    for st in [16 * g for g in range(RPT // 16)] + [RPT - 16]:
        sl = pl.ds(st, 16)
        d = dva[sl] + dvb[sl]
        yi = jnp.int32(0x5F3759DF) - lax.shift_right_logical(
            plsc.bitcast(d, jnp.int32), 1)
        y = plsc.bitcast(yi, jnp.float32)
        h = d * 0.5
        y = y * (1.5 - h * y * y)
        y = y * (1.5 - h * y * y)
        y = y * (1.5 - h * y * y)
        dinv_v[sl] = jnp.where(d > 0, y, 0.0)

    def _pass(xq, msgq, q):
        def _zrow(i, _):
            for v in range(H // 16):
                dbuf[i, pl.ds(16 * v, 16)] = zeros16
            return 0
        lax.fori_loop(0, CH, _zrow, 0)
        for k in range(RPT // CH):
            pltpu.sync_copy(dbuf, acc.at[pl.ds(base + k * CH, CH), :])
        pltpu.sync_copy(bconv.at[pl.ds(H * q, H)], bcv)
        plsc.subcore_barrier()

        rbufs = (rb0, rb1, rb2, rb3)
        sgs = (sg0, sg1, sg2, sg3)
        sss = (ss0, ss1, ss2, ss3)
        pltpu.async_copy(xq.at[rowv.at[0]], rbufs[0], sgs[0])
        pltpu.async_copy(xq.at[rowv.at[1]], rbufs[1], sgs[1])

        def _slot(g, _):
            for b in range(4):
                j = 4 * g + b
                b2 = (b + 2) % 4

                @pl.when(j >= 2)
                def _():
                    pltpu.make_async_copy(
                        rbufs[b2], acc.at[colv.at[j - 2]], sss[b2]).wait()

                @pl.when(j + 2 < K3C)
                def _():
                    pltpu.async_copy(
                        xq.at[rowv.at[j + 2]], rbufs[b2], sgs[b2])

                pltpu.make_async_copy(
                    xq.at[rowv.at[j]], rbufs[b], sgs[b]).wait()
                pltpu.async_copy(
                    rbufs[b], acc.at[colv.at[j]], sss[b], add=True)
            return 0
        lax.fori_loop(0, K3C // 4, _slot, 0)
        pltpu.make_async_copy(
            rbufs[(K3C - 2) % 4], acc.at[colv.at[K3C - 2]],
            sss[(K3C - 2) % 4]).wait()
        pltpu.make_async_copy(
            rbufs[(K3C - 1) % 4], acc.at[colv.at[K3C - 1]],
            sss[(K3C - 1) % 4]).wait()
        plsc.subcore_barrier()

        for k in range(RPT // CH):
            r0 = base + k * CH
            pltpu.sync_copy(acc.at[pl.ds(r0, CH), :], dbuf)

            def _scale(i, _):
                dv = plsc.load_gather(
                    dinv_v, [jnp.full((16,), k * CH + i, jnp.int32)])
                for v in range(H // 16):
                    sl = pl.ds(16 * v, 16)
                    dbuf[i, sl] = dbuf[i, sl] * dv + bcv[sl]
                return 0
            lax.fori_loop(0, CH, _scale, 0)
            pltpu.sync_copy(dbuf, msgq.at[pl.ds(r0, CH), :])

    @pl.when(c == 0)
    def _():
        _pass(x0, msg0, 0)
        _pass(x1, msg1, 1)

    @pl.when(c == 1)
    def _():
        _pass(x2, msg2, 2)
        _pass(x3, msg3, 3)


@functools.cache
def _msg_call():
    return pl.kernel(
        _msg_body,
        out_type=tuple(jax.ShapeDtypeStruct((N, H), jnp.float32)
                       for _ in range(4)),
        mesh=_mesh(),
        scratch_types=[
            pltpu.VMEM((K3C, CH), jnp.int32),
            pltpu.VMEM((K3C, CH), jnp.int32),
            pltpu.VMEM((CH, H), jnp.float32),
            pltpu.VMEM((CH, H), jnp.float32),
            pltpu.VMEM((CH, H), jnp.float32),
            pltpu.VMEM((CH, H), jnp.float32),
            pltpu.VMEM((CH, H), jnp.float32),
            pltpu.VMEM((RPT,), jnp.float32),
            pltpu.VMEM((RPT,), jnp.float32),
            pltpu.VMEM((RPT,), jnp.float32),
            pltpu.VMEM((H,), jnp.float32),
            pltpu.SemaphoreType.DMA,
            pltpu.SemaphoreType.DMA,
            pltpu.SemaphoreType.DMA,
            pltpu.SemaphoreType.DMA,
            pltpu.SemaphoreType.DMA,
            pltpu.SemaphoreType.DMA,
            pltpu.SemaphoreType.DMA,
            pltpu.SemaphoreType.DMA,
            pltpu.VMEM_SHARED((N, H), jnp.float32),
        ],
        compiler_params=pltpu.CompilerParams(use_tc_tiling_on_sc=False, needs_layout_passes=False),
    )


def kernel(edge_index, batch, sphex, W_conv, b_conv, W_lin, b_lin):
    del batch
    row = edge_index[0]
    col = edge_index[1]
    col_k1 = col.reshape(NC, NS, K1C, CH)
    row_k3 = row.reshape(NS, K3C, CH)
    col_k3 = col.reshape(NS, K3C, CH)

    deg_parts = _deg_call()(col_k1)

    sphex_pad = jnp.pad(sphex, ((0, 0), (0, 1)))
    deg_blocked = deg_parts.reshape(NC, NBLK, BN).transpose(1, 0, 2)
    x0, x1, x2, x3, msg_intra, logz = _dense_call(
        sphex_pad, W_conv, W_lin, b_lin.reshape(1, D), deg_blocked)

    msg0, msg1, msg2, msg3 = _msg_call()(
        row_k3, col_k3, x0, x1, x2, x3, deg_parts, b_conv)
    msg = jnp.concatenate([msg0, msg1, msg2, msg3], axis=1)
    return (msg, msg_intra, logz)


# speedup vs baseline: 13.3257x; 1.0372x over previous
import functools

import jax
import jax.numpy as jnp
from jax import lax
from jax.experimental import pallas as pl
from jax.experimental.pallas import tpu as pltpu
from jax.experimental.pallas import tpu_sc as plsc

N = 10000
E = 160000
D = 256
H = D // 4
NNB = 16
NC = 2
NS = 16
CH = 125
K1C = E // (NC * NS * CH)
K3C = E // (NS * CH)
GRP = 2
K3G = K3C // GRP
RPT = N // NS
BN = 1000
NBLK = N // BN

_mesh = functools.partial(
    plsc.VectorSubcoreMesh, core_axis_name="c", subcore_axis_name="s",
    num_cores=NC, num_subcores=NS)


def _deg_body(col3, deg_out, colv, ones_v, zero_v, dchunk, dcomp, deg_sp):
    c = lax.axis_index("c")
    s = lax.axis_index("s")

    ones16 = jnp.ones((16,), jnp.float32)
    zeros16 = jnp.zeros((16,), jnp.float32)

    def _fill(i, _):
        ones_v[i, :] = ones16
        zero_v[i, :] = zeros16
        return 0
    lax.fori_loop(0, CH, _fill, 0)

    for k in range(RPT // CH):
        pltpu.sync_copy(zero_v, deg_sp.at[pl.ds(s * RPT + k * CH, CH), :])
    plsc.subcore_barrier()

    pltpu.sync_copy(col3.at[c, s], colv)

    def _scat(j, _):
        pltpu.sync_copy(ones_v, deg_sp.at[colv.at[j]], add=True)
        return 0
    lax.fori_loop(0, K1C, _scat, 0)
    plsc.subcore_barrier()

    pltpu.sync_copy(deg_sp.at[pl.ds(s * RPT, RPT), :], dchunk)
    i16 = lax.broadcasted_iota(jnp.int32, (16,), 0)
    z16 = jnp.zeros((16,), jnp.int32)
    for st in [16 * g for g in range(RPT // 16)] + [RPT - 16]:
        dcomp[pl.ds(st, 16)] = plsc.load_gather(dchunk, [i16 + st, z16])
    pltpu.sync_copy(dcomp, deg_out.at[c, s])


@functools.cache
def _deg_call():
    return pl.kernel(
        _deg_body,
        out_type=jax.ShapeDtypeStruct((NC, NS, RPT), jnp.float32),
        mesh=_mesh(),
        scratch_types=[
            pltpu.VMEM((K1C, CH), jnp.int32),
            pltpu.VMEM((CH, 16), jnp.float32),
            pltpu.VMEM((CH, 16), jnp.float32),
            pltpu.VMEM((RPT, 16), jnp.float32),
            pltpu.VMEM((RPT,), jnp.float32),
            pltpu.VMEM_SHARED((N, 16), jnp.float32),
        ],
        compiler_params=pltpu.CompilerParams(use_tc_tiling_on_sc=False, needs_layout_passes=False),
    )


def _dense_body(sphex_ref, wc_ref, wl_ref, bl_ref, deg_ref,
                x0_ref, x1_ref, x2_ref, x3_ref, mi_ref, logz_ref,
                acc_ref):
    k = pl.program_id(0)
    f32 = jnp.float32

    sp = sphex_ref[...]
    sin = jnp.sin(sp)
    cos = jnp.cos(sp)
    u = jnp.log(jnp.maximum(jnp.abs(sin), 1e-30))
    neg = jnp.where(sin < 0, 1.0, 0.0).astype(f32)

    jj = lax.broadcasted_iota(jnp.int32, (D, D), 0)
    ii = lax.broadcasted_iota(jnp.int32, (D, D), 1)
    M = jnp.where(jj < ii, 1.0, 0.0).astype(f32)

    dn = (((1,), (0,)), ((), ()))
    prefix_log = lax.dot_general(u, M, dn, preferred_element_type=f32)
    negcnt = lax.dot_general(neg, M, dn, preferred_element_type=f32)
    parity = negcnt - 2.0 * jnp.floor(negcnt * 0.5)
    sign = 1.0 - 2.0 * parity
    colid = lax.broadcasted_iota(jnp.int32, (BN, D), 1)
    cos_part = jnp.where(colid == D - 1, 1.0, cos)
    gex = sign * jnp.exp(prefix_log) * cos_part

    dnt = (((1,), (1,)), ((), ()))
    x = lax.dot_general(gex, wc_ref[...], dnt, preferred_element_type=f32)
    dblk = deg_ref[0]
    deg = dblk[0:1, :] + dblk[1:2, :]
    dinv_row = jnp.where(deg > 0, lax.rsqrt(jnp.maximum(deg, 1e-12)), 0.0)
    dinv = lax.dot_general(dinv_row, jnp.ones((1, 1), f32),
                           (((0,), (0,)), ((), ())),
                           preferred_element_type=f32)
    xp = x * dinv
    x0_ref[...] = xp[:, :H]
    x1_ref[...] = xp[:, H:2 * H]
    x2_ref[...] = xp[:, 2 * H:3 * H]
    x3_ref[...] = xp[:, 3 * H:]
    mi_ref[...] = (lax.dot_general(gex, wl_ref[...], dnt,
                                   preferred_element_type=f32) + bl_ref[...])

    colsum = jnp.sum(gex, axis=0, keepdims=True)

    @pl.when(k == 0)
    def _():
        acc_ref[...] = colsum

    @pl.when(k > 0)
    def _():
        acc_ref[...] = acc_ref[...] + colsum

    @pl.when(k == NBLK - 1)
    def _():
        m = acc_ref[...] * (1.0 / N)
        A = NNB * wc_ref[...] + 2.0 * wl_ref[...]
        v = lax.dot_general(A, m, (((1,), (1,)), ((), ())),
                            preferred_element_type=f32)
        g = jnp.sqrt(jnp.sum(v * v))
        B = wl_ref[...] + 0.5 * NNB * wc_ref[...]
        t = lax.dot_general(m, B, (((1,), (0,)), ((), ())),
                            preferred_element_type=f32)
        z_mean = -float(N) * jnp.sum(t * m)
        g_hi = jnp.maximum(g, 20.0)
        g_lo = jnp.minimum(g, 20.0)
        z_hi = float(N) * (g_hi - jnp.log(g_hi))
        z_lo = float(N) * jnp.log(
            (jnp.exp(g_lo) - jnp.exp(-g_lo)) / jnp.maximum(g_lo, 1e-30))
        z_int = jnp.where(g > 20.0, z_hi, z_lo)
        logz_ref[...] = jnp.full((1, 1), 0.0, f32) + z_mean + z_int


_dense_call = pl.pallas_call(
    _dense_body,
    grid=(NBLK,),
    in_specs=[
        pl.BlockSpec((BN, D), lambda k: (k, 0)),
        pl.BlockSpec((D, D), lambda k: (0, 0)),
        pl.BlockSpec((D, D), lambda k: (0, 0)),
        pl.BlockSpec((1, D), lambda k: (0, 0)),
        pl.BlockSpec((1, 2, BN), lambda k: (k, 0, 0)),
    ],
    out_specs=[
        pl.BlockSpec((BN, H), lambda k: (k, 0)),
        pl.BlockSpec((BN, H), lambda k: (k, 0)),
        pl.BlockSpec((BN, H), lambda k: (k, 0)),
        pl.BlockSpec((BN, H), lambda k: (k, 0)),
        pl.BlockSpec((BN, D), lambda k: (k, 0)),
        pl.BlockSpec((1, 1), lambda k: (0, 0)),
    ],
    out_shape=[
        jax.ShapeDtypeStruct((N, H), jnp.float32),
        jax.ShapeDtypeStruct((N, H), jnp.float32),
        jax.ShapeDtypeStruct((N, H), jnp.float32),
        jax.ShapeDtypeStruct((N, H), jnp.float32),
        jax.ShapeDtypeStruct((N, D), jnp.float32),
        jax.ShapeDtypeStruct((1, 1), jnp.float32),
    ],
    scratch_shapes=[pltpu.VMEM((1, D), jnp.float32)],
)


def _msg_body(row3, col3, x0, x1, x2, x3, degc, bconv, msg0, msg1, msg2, msg3,
              rowv, colv, rb0, rb1, rb2, dbuf, dva, dvb, dinv_v, bcv,
              sg0, sg1, sg2, ss0, ss1, ss2, acc):
    c = lax.axis_index("c")
    s = lax.axis_index("s")
    base = s * RPT

    zeros16 = jnp.zeros((16,), jnp.float32)

    pltpu.sync_copy(row3.at[s], rowv)
    pltpu.sync_copy(col3.at[s], colv)

    pltpu.sync_copy(degc.at[0, s], dva)
    pltpu.sync_copy(degc.at[1, s], dvb)
    for st in [16 * g for g in range(RPT // 16)] + [RPT - 16]:
        sl = pl.ds(st, 16)
        d = dva[sl] + dvb[sl]
        yi = jnp.int32(0x5F3759DF) - lax.shift_right_logical(
            plsc.bitcast(d, jnp.int32), 1)
        y = plsc.bitcast(yi, jnp.float32)
        h = d * 0.5
        y = y * (1.5 - h * y * y)
        y = y * (1.5 - h * y * y)
        y = y * (1.5 - h * y * y)
        dinv_v[sl] = jnp.where(d > 0, y, 0.0)

    def _pass(xq, msgq, q):
        def _zrow(i, _):
            for v in range(H // 16):
                dbuf[i, pl.ds(16 * v, 16)] = zeros16
            return 0
        lax.fori_loop(0, CH, _zrow, 0)
        for k in range(RPT // CH):
            pltpu.sync_copy(dbuf, acc.at[pl.ds(base + k * CH, CH), :])
        pltpu.sync_copy(bconv.at[pl.ds(H * q, H)], bcv)
        plsc.subcore_barrier()

        rbufs = (rb0, rb1, rb2)
        sgs = (sg0, sg1, sg2)
        sss = (ss0, ss1, ss2)
        pltpu.async_copy(xq.at[rowv.at[0]], rbufs[0], sgs[0])
        pltpu.async_copy(xq.at[rowv.at[1]], rbufs[1], sgs[1])

        def _slot(g, _):
            for b in range(3):
                j = 3 * g + b
                b2 = (b + 2) % 3

                @pl.when(j >= 1)
                def _():
                    pltpu.make_async_copy(
                        rbufs[b2], acc.at[colv.at[j - 1]], sss[b2]).wait()

                @pl.when(j + 2 < K3G)
                def _():
                    pltpu.async_copy(
                        xq.at[rowv.at[j + 2]], rbufs[b2], sgs[b2])

                pltpu.make_async_copy(
                    xq.at[rowv.at[j]], rbufs[b], sgs[b]).wait()
                pltpu.async_copy(
                    rbufs[b], acc.at[colv.at[j]], sss[b], add=True)
            return 0
        lax.fori_loop(0, K3G // 3, _slot, 0)
        for j in range(3 * (K3G // 3), K3G):
            b = j % 3

            @pl.when(j >= 1)
            def _():
                pltpu.make_async_copy(
                    rbufs[(b + 2) % 3], acc.at[colv.at[j - 1]],
                    sss[(b + 2) % 3]).wait()

            pltpu.make_async_copy(xq.at[rowv.at[j]], rbufs[b], sgs[b]).wait()
            pltpu.async_copy(rbufs[b], acc.at[colv.at[j]], sss[b], add=True)
        pltpu.make_async_copy(
            rbufs[(K3G - 1) % 3], acc.at[colv.at[K3G - 1]],
            sss[(K3G - 1) % 3]).wait()
        plsc.subcore_barrier()

        for k in range(RPT // CH):
            r0 = base + k * CH
            pltpu.sync_copy(acc.at[pl.ds(r0, CH), :], dbuf)

            def _scale(i, _):
                dv = plsc.load_gather(
                    dinv_v, [jnp.full((16,), k * CH + i, jnp.int32)])
                for v in range(H // 16):
                    sl = pl.ds(16 * v, 16)
                    dbuf[i, sl] = dbuf[i, sl] * dv + bcv[sl]
                return 0
            lax.fori_loop(0, CH, _scale, 0)
            pltpu.sync_copy(dbuf, msgq.at[pl.ds(r0, CH), :])

    @pl.when(c == 0)
    def _():
        _pass(x0, msg0, 0)
        _pass(x1, msg1, 1)

    @pl.when(c == 1)
    def _():
        _pass(x2, msg2, 2)
        _pass(x3, msg3, 3)


@functools.cache
def _msg_call():
    return pl.kernel(
        _msg_body,
        out_type=tuple(jax.ShapeDtypeStruct((N, H), jnp.float32)
                       for _ in range(4)),
        mesh=_mesh(),
        scratch_types=[
            pltpu.VMEM((K3G, GRP * CH), jnp.int32),
            pltpu.VMEM((K3G, GRP * CH), jnp.int32),
            pltpu.VMEM((GRP * CH, H), jnp.float32),
            pltpu.VMEM((GRP * CH, H), jnp.float32),
            pltpu.VMEM((GRP * CH, H), jnp.float32),
            pltpu.VMEM((CH, H), jnp.float32),
            pltpu.VMEM((RPT,), jnp.float32),
            pltpu.VMEM((RPT,), jnp.float32),
            pltpu.VMEM((RPT,), jnp.float32),
            pltpu.VMEM((H,), jnp.float32),
            pltpu.SemaphoreType.DMA,
            pltpu.SemaphoreType.DMA,
            pltpu.SemaphoreType.DMA,
            pltpu.SemaphoreType.DMA,
            pltpu.SemaphoreType.DMA,
            pltpu.SemaphoreType.DMA,
            pltpu.VMEM_SHARED((N, H), jnp.float32),
        ],
        compiler_params=pltpu.CompilerParams(use_tc_tiling_on_sc=False, needs_layout_passes=False),
    )


def kernel(edge_index, batch, sphex, W_conv, b_conv, W_lin, b_lin):
    del batch
    row = edge_index[0]
    col = edge_index[1]
    col_k1 = col.reshape(NC, NS, K1C, CH)
    row_k3 = row.reshape(NS, K3G, GRP * CH)
    col_k3 = col.reshape(NS, K3G, GRP * CH)

    deg_parts = _deg_call()(col_k1)

    sphex_pad = jnp.pad(sphex, ((0, 0), (0, 1)))
    deg_blocked = deg_parts.reshape(NC, NBLK, BN).transpose(1, 0, 2)
    x0, x1, x2, x3, msg_intra, logz = _dense_call(
        sphex_pad, W_conv, W_lin, b_lin.reshape(1, D), deg_blocked)

    msg0, msg1, msg2, msg3 = _msg_call()(
        row_k3, col_k3, x0, x1, x2, x3, deg_parts, b_conv)
    msg = jnp.concatenate([msg0, msg1, msg2, msg3], axis=1)
    return (msg, msg_intra, logz)


# speedup vs baseline: 13.4181x; 1.0069x over previous
import functools

import jax
import jax.numpy as jnp
from jax import lax
from jax.experimental import pallas as pl
from jax.experimental.pallas import tpu as pltpu
from jax.experimental.pallas import tpu_sc as plsc

N = 10000
E = 160000
D = 256
H = D // 4
NNB = 16
NC = 2
NS = 16
CH = 125
K1C = E // (NC * NS * CH)
K3C = E // (NS * CH)
GRP = 2
K3G = K3C // GRP
RPT = N // NS
BN = 2000
NBLK = N // BN

_mesh = functools.partial(
    plsc.VectorSubcoreMesh, core_axis_name="c", subcore_axis_name="s",
    num_cores=NC, num_subcores=NS)


def _deg_body(col3, deg_out, colv, ones_v, zero_v, dchunk, dcomp, deg_sp):
    c = lax.axis_index("c")
    s = lax.axis_index("s")

    ones16 = jnp.ones((16,), jnp.float32)
    zeros16 = jnp.zeros((16,), jnp.float32)

    def _fill(i, _):
        ones_v[i, :] = ones16
        zero_v[i, :] = zeros16
        return 0
    lax.fori_loop(0, CH, _fill, 0)

    for k in range(RPT // CH):
        pltpu.sync_copy(zero_v, deg_sp.at[pl.ds(s * RPT + k * CH, CH), :])
    plsc.subcore_barrier()

    pltpu.sync_copy(col3.at[c, s], colv)

    def _scat(j, _):
        pltpu.sync_copy(ones_v, deg_sp.at[colv.at[j]], add=True)
        return 0
    lax.fori_loop(0, K1C, _scat, 0)
    plsc.subcore_barrier()

    pltpu.sync_copy(deg_sp.at[pl.ds(s * RPT, RPT), :], dchunk)
    i16 = lax.broadcasted_iota(jnp.int32, (16,), 0)
    z16 = jnp.zeros((16,), jnp.int32)
    for st in [16 * g for g in range(RPT // 16)] + [RPT - 16]:
        dcomp[pl.ds(st, 16)] = plsc.load_gather(dchunk, [i16 + st, z16])
    pltpu.sync_copy(dcomp, deg_out.at[c, s])


@functools.cache
def _deg_call():
    return pl.kernel(
        _deg_body,
        out_type=jax.ShapeDtypeStruct((NC, NS, RPT), jnp.float32),
        mesh=_mesh(),
        scratch_types=[
            pltpu.VMEM((K1C, CH), jnp.int32),
            pltpu.VMEM((CH, 16), jnp.float32),
            pltpu.VMEM((CH, 16), jnp.float32),
            pltpu.VMEM((RPT, 16), jnp.float32),
            pltpu.VMEM((RPT,), jnp.float32),
            pltpu.VMEM_SHARED((N, 16), jnp.float32),
        ],
        compiler_params=pltpu.CompilerParams(use_tc_tiling_on_sc=False, needs_layout_passes=False),
    )


def _dense_body(sphex_ref, wc_ref, wl_ref, bl_ref, deg_ref,
                x0_ref, x1_ref, x2_ref, x3_ref, mi_ref, logz_ref,
                acc_ref):
    k = pl.program_id(0)
    f32 = jnp.float32

    sp = sphex_ref[...]
    sin = jnp.sin(sp)
    cos = jnp.cos(sp)
    u = jnp.log(jnp.maximum(jnp.abs(sin), 1e-30))
    neg = jnp.where(sin < 0, 1.0, 0.0).astype(f32)

    jj = lax.broadcasted_iota(jnp.int32, (D, D), 0)
    ii = lax.broadcasted_iota(jnp.int32, (D, D), 1)
    M = jnp.where(jj < ii, 1.0, 0.0).astype(f32)

    dn = (((1,), (0,)), ((), ()))
    prefix_log = lax.dot_general(u, M, dn, preferred_element_type=f32)
    negcnt = lax.dot_general(neg, M, dn, preferred_element_type=f32)
    parity = negcnt - 2.0 * jnp.floor(negcnt * 0.5)
    sign = 1.0 - 2.0 * parity
    colid = lax.broadcasted_iota(jnp.int32, (BN, D), 1)
    cos_part = jnp.where(colid == D - 1, 1.0, cos)
    gex = sign * jnp.exp(prefix_log) * cos_part

    dnt = (((1,), (1,)), ((), ()))
    x = lax.dot_general(gex, wc_ref[...], dnt, preferred_element_type=f32)
    dblk = deg_ref[0]
    deg = dblk[0:1, :] + dblk[1:2, :]
    dinv_row = jnp.where(deg > 0, lax.rsqrt(jnp.maximum(deg, 1e-12)), 0.0)
    dinv = lax.dot_general(dinv_row, jnp.ones((1, 1), f32),
                           (((0,), (0,)), ((), ())),
                           preferred_element_type=f32)
    xp = x * dinv
    x0_ref[...] = xp[:, :H]
    x1_ref[...] = xp[:, H:2 * H]
    x2_ref[...] = xp[:, 2 * H:3 * H]
    x3_ref[...] = xp[:, 3 * H:]
    mi_ref[...] = (lax.dot_general(gex, wl_ref[...], dnt,
                                   preferred_element_type=f32) + bl_ref[...])

    colsum = jnp.sum(gex, axis=0, keepdims=True)

    @pl.when(k == 0)
    def _():
        acc_ref[...] = colsum

    @pl.when(k > 0)
    def _():
        acc_ref[...] = acc_ref[...] + colsum

    @pl.when(k == NBLK - 1)
    def _():
        m = acc_ref[...] * (1.0 / N)
        A = NNB * wc_ref[...] + 2.0 * wl_ref[...]
        v = lax.dot_general(A, m, (((1,), (1,)), ((), ())),
                            preferred_element_type=f32)
        g = jnp.sqrt(jnp.sum(v * v))
        B = wl_ref[...] + 0.5 * NNB * wc_ref[...]
        t = lax.dot_general(m, B, (((1,), (0,)), ((), ())),
                            preferred_element_type=f32)
        z_mean = -float(N) * jnp.sum(t * m)
        g_hi = jnp.maximum(g, 20.0)
        g_lo = jnp.minimum(g, 20.0)
        z_hi = float(N) * (g_hi - jnp.log(g_hi))
        z_lo = float(N) * jnp.log(
            (jnp.exp(g_lo) - jnp.exp(-g_lo)) / jnp.maximum(g_lo, 1e-30))
        z_int = jnp.where(g > 20.0, z_hi, z_lo)
        logz_ref[...] = jnp.full((1, 1), 0.0, f32) + z_mean + z_int


_dense_call = pl.pallas_call(
    _dense_body,
    grid=(NBLK,),
    in_specs=[
        pl.BlockSpec((BN, D), lambda k: (k, 0)),
        pl.BlockSpec((D, D), lambda k: (0, 0)),
        pl.BlockSpec((D, D), lambda k: (0, 0)),
        pl.BlockSpec((1, D), lambda k: (0, 0)),
        pl.BlockSpec((1, 2, BN), lambda k: (k, 0, 0)),
    ],
    out_specs=[
        pl.BlockSpec((BN, H), lambda k: (k, 0)),
        pl.BlockSpec((BN, H), lambda k: (k, 0)),
        pl.BlockSpec((BN, H), lambda k: (k, 0)),
        pl.BlockSpec((BN, H), lambda k: (k, 0)),
        pl.BlockSpec((BN, D), lambda k: (k, 0)),
        pl.BlockSpec((1, 1), lambda k: (0, 0)),
    ],
    out_shape=[
        jax.ShapeDtypeStruct((N, H), jnp.float32),
        jax.ShapeDtypeStruct((N, H), jnp.float32),
        jax.ShapeDtypeStruct((N, H), jnp.float32),
        jax.ShapeDtypeStruct((N, H), jnp.float32),
        jax.ShapeDtypeStruct((N, D), jnp.float32),
        jax.ShapeDtypeStruct((1, 1), jnp.float32),
    ],
    scratch_shapes=[pltpu.VMEM((1, D), jnp.float32)],
)


def _msg_body(row3, col3, x0, x1, x2, x3, degc, bconv, msg0, msg1, msg2, msg3,
              rowv, colv, rb0, rb1, rb2, dbuf, dva, dvb, dinv_v, bcv,
              sg0, sg1, sg2, ss0, ss1, ss2, acc):
    c = lax.axis_index("c")
    s = lax.axis_index("s")
    base = s * RPT

    zeros16 = jnp.zeros((16,), jnp.float32)

    pltpu.sync_copy(row3.at[s], rowv)
    pltpu.sync_copy(col3.at[s], colv)

    pltpu.sync_copy(degc.at[0, s], dva)
    pltpu.sync_copy(degc.at[1, s], dvb)
    for st in [16 * g for g in range(RPT // 16)] + [RPT - 16]:
        sl = pl.ds(st, 16)
        d = dva[sl] + dvb[sl]
        yi = jnp.int32(0x5F3759DF) - lax.shift_right_logical(
            plsc.bitcast(d, jnp.int32), 1)
        y = plsc.bitcast(yi, jnp.float32)
        h = d * 0.5
        y = y * (1.5 - h * y * y)
        y = y * (1.5 - h * y * y)
        y = y * (1.5 - h * y * y)
        dinv_v[sl] = jnp.where(d > 0, y, 0.0)

    def _pass(xq, msgq, q):
        def _zrow(i, _):
            for v in range(H // 16):
                dbuf[i, pl.ds(16 * v, 16)] = zeros16
            return 0
        lax.fori_loop(0, CH, _zrow, 0)
        for k in range(RPT // CH):
            pltpu.sync_copy(dbuf, acc.at[pl.ds(base + k * CH, CH), :])
        pltpu.sync_copy(bconv.at[pl.ds(H * q, H)], bcv)
        plsc.subcore_barrier()

        rbufs = (rb0, rb1, rb2)
        sgs = (sg0, sg1, sg2)
        sss = (ss0, ss1, ss2)
        pltpu.async_copy(xq.at[rowv.at[0]], rbufs[0], sgs[0])
        pltpu.async_copy(xq.at[rowv.at[1]], rbufs[1], sgs[1])

        def _slot(g, _):
            for b in range(3):
                j = 3 * g + b
                b2 = (b + 2) % 3

                @pl.when(j >= 1)
                def _():
                    pltpu.make_async_copy(
                        rbufs[b2], acc.at[colv.at[j - 1]], sss[b2]).wait()

                @pl.when(j + 2 < K3G)
                def _():
                    pltpu.async_copy(
                        xq.at[rowv.at[j + 2]], rbufs[b2], sgs[b2])

                pltpu.make_async_copy(
                    xq.at[rowv.at[j]], rbufs[b], sgs[b]).wait()
                pltpu.async_copy(
                    rbufs[b], acc.at[colv.at[j]], sss[b], add=True)
            return 0
        lax.fori_loop(0, K3G // 3, _slot, 0)
        for j in range(3 * (K3G // 3), K3G):
            b = j % 3

            @pl.when(j >= 1)
            def _():
                pltpu.make_async_copy(
                    rbufs[(b + 2) % 3], acc.at[colv.at[j - 1]],
                    sss[(b + 2) % 3]).wait()

            pltpu.make_async_copy(xq.at[rowv.at[j]], rbufs[b], sgs[b]).wait()
            pltpu.async_copy(rbufs[b], acc.at[colv.at[j]], sss[b], add=True)
        pltpu.make_async_copy(
            rbufs[(K3G - 1) % 3], acc.at[colv.at[K3G - 1]],
            sss[(K3G - 1) % 3]).wait()
        plsc.subcore_barrier()

        for k in range(RPT // CH):
            r0 = base + k * CH
            pltpu.sync_copy(acc.at[pl.ds(r0, CH), :], dbuf)

            def _scale(i, _):
                dv = plsc.load_gather(
                    dinv_v, [jnp.full((16,), k * CH + i, jnp.int32)])
                for v in range(H // 16):
                    sl = pl.ds(16 * v, 16)
                    dbuf[i, sl] = dbuf[i, sl] * dv + bcv[sl]
                return 0
            lax.fori_loop(0, CH, _scale, 0)
            pltpu.sync_copy(dbuf, msgq.at[pl.ds(r0, CH), :])

    @pl.when(c == 0)
    def _():
        _pass(x0, msg0, 0)
        _pass(x1, msg1, 1)

    @pl.when(c == 1)
    def _():
        _pass(x2, msg2, 2)
        _pass(x3, msg3, 3)


@functools.cache
def _msg_call():
    return pl.kernel(
        _msg_body,
        out_type=tuple(jax.ShapeDtypeStruct((N, H), jnp.float32)
                       for _ in range(4)),
        mesh=_mesh(),
        scratch_types=[
            pltpu.VMEM((K3G, GRP * CH), jnp.int32),
            pltpu.VMEM((K3G, GRP * CH), jnp.int32),
            pltpu.VMEM((GRP * CH, H), jnp.float32),
            pltpu.VMEM((GRP * CH, H), jnp.float32),
            pltpu.VMEM((GRP * CH, H), jnp.float32),
            pltpu.VMEM((CH, H), jnp.float32),
            pltpu.VMEM((RPT,), jnp.float32),
            pltpu.VMEM((RPT,), jnp.float32),
            pltpu.VMEM((RPT,), jnp.float32),
            pltpu.VMEM((H,), jnp.float32),
            pltpu.SemaphoreType.DMA,
            pltpu.SemaphoreType.DMA,
            pltpu.SemaphoreType.DMA,
            pltpu.SemaphoreType.DMA,
            pltpu.SemaphoreType.DMA,
            pltpu.SemaphoreType.DMA,
            pltpu.VMEM_SHARED((N, H), jnp.float32),
        ],
        compiler_params=pltpu.CompilerParams(use_tc_tiling_on_sc=False, needs_layout_passes=False),
    )


def kernel(edge_index, batch, sphex, W_conv, b_conv, W_lin, b_lin):
    del batch
    row = edge_index[0]
    col = edge_index[1]
    col_k1 = col.reshape(NC, NS, K1C, CH)
    row_k3 = row.reshape(NS, K3G, GRP * CH)
    col_k3 = col.reshape(NS, K3G, GRP * CH)

    deg_parts = _deg_call()(col_k1)

    sphex_pad = jnp.pad(sphex, ((0, 0), (0, 1)))
    deg_blocked = deg_parts.reshape(NC, NBLK, BN).transpose(1, 0, 2)
    x0, x1, x2, x3, msg_intra, logz = _dense_call(
        sphex_pad, W_conv, W_lin, b_lin.reshape(1, D), deg_blocked)

    msg0, msg1, msg2, msg3 = _msg_call()(
        row_k3, col_k3, x0, x1, x2, x3, deg_parts, b_conv)
    msg = jnp.concatenate([msg0, msg1, msg2, msg3], axis=1)
    return (msg, msg_intra, logz)
